# Initial kernel scaffold; baseline (speedup 1.0000x reference)
import jax
import jax.numpy as jnp
from jax.experimental import pallas as pl


def kernel(x, edge_index, W1, b1, gamma, beta, W2, b2):
    raise NotImplementedError("write your pallas kernel here")


# speedup vs baseline: 26.4496x; 26.4496x over previous
import functools

import jax
import jax.numpy as jnp
from jax import lax
from jax.experimental import pallas as pl
from jax.experimental.pallas import tpu as pltpu
from jax.experimental.pallas import tpu_sc as plsc

N_NODES = 10000
N_EDGES = 320000
BN_EPS = 1e-5

_NC = 2
_NS = 16
_NW = _NC * _NS
_L = 16


def _deg_body(col_hbm, out_hbm, colbuf, degbuf):
    wid = lax.axis_index("s") * _NC + lax.axis_index("c")
    epw = N_EDGES // _NW
    pltpu.sync_copy(col_hbm.at[pl.ds(wid * epw, epw)], colbuf)

    zeros16 = jnp.zeros((_L,), jnp.float32)
    ones16 = jnp.ones((_L,), jnp.float32)

    def _zero(i, _):
        degbuf[pl.ds(i * _L, _L)] = zeros16
        return ()

    lax.fori_loop(0, N_NODES // _L, _zero, (), unroll=8)

    def _hist(i, _):
        idx = colbuf[pl.ds(i * _L, _L)]
        plsc.addupdate_scatter(degbuf, [idx], ones16)
        return ()

    lax.fori_loop(0, epw // _L, _hist, (), unroll=8)
    pltpu.sync_copy(degbuf, out_hbm.at[wid])


@functools.partial(jax.jit, static_argnames=())
def _deg_partials(col):
    mesh = plsc.VectorSubcoreMesh(core_axis_name="c", subcore_axis_name="s")
    return pl.kernel(
        _deg_body,
        out_type=jax.ShapeDtypeStruct((_NW, N_NODES), jnp.float32),
        mesh=mesh,
        scratch_types=[
            pltpu.VMEM((N_EDGES // _NW,), jnp.int32),
            pltpu.VMEM((N_NODES,), jnp.float32),
        ],
        compiler_params=pltpu.CompilerParams(needs_layout_passes=False),
    )(col)


_CHUNK = 125
_EPW = N_EDGES // _NW
_NCHUNK = _EPW // _CHUNK
_ZROWS = 128
_NPAD = 10240
_RPT = _NPAD // _NS


def _make_agg(D, tc_tiling=True):
    def body(g_hbm, row_hbm, col_hbm, out_hbm, rowbuf, colbuf, msg, acc, gsem):
        c = lax.axis_index("c")
        s = lax.axis_index("s")
        wid = s * _NC + c

        pltpu.sync_copy(row_hbm.at[wid], rowbuf)
        pltpu.sync_copy(col_hbm.at[wid], colbuf)

        zeros16 = jnp.zeros((_L,), jnp.float32)

        def _zero(i, _):
            msg[i // (D // _L), pl.ds((i % (D // _L)) * _L, _L)] = zeros16
            return ()

        lax.fori_loop(0, _ZROWS * D // _L, _zero, (), unroll=8)

        for j in range(_RPT // _ZROWS):
            pltpu.sync_copy(msg, acc.at[pl.ds(s * _RPT + j * _ZROWS, _ZROWS)])
        plsc.subcore_barrier()

        gdst = msg.at[pl.ds(0, _CHUNK)]

        def _step(k, _):
            pltpu.async_copy(g_hbm.at[rowbuf.at[k]], gdst, gsem).wait()
            pltpu.sync_copy(gdst, acc.at[colbuf.at[k]], add=True)
            return ()

        lax.fori_loop(0, _NCHUNK, _step, ())
        plsc.subcore_barrier()
        pltpu.sync_copy(acc.at[pl.ds(s * _RPT, _RPT)],
                        out_hbm.at[c, pl.ds(s * _RPT, _RPT)])

    mesh = plsc.VectorSubcoreMesh(core_axis_name="c", subcore_axis_name="s")
    return pl.kernel(
        body,
        out_type=jax.ShapeDtypeStruct((_NC, _NPAD, D), jnp.float32),
        mesh=mesh,
        scratch_types=[
            pltpu.VMEM((_NCHUNK, _CHUNK), jnp.int32),
            pltpu.VMEM((_NCHUNK, _CHUNK), jnp.int32),
            pltpu.VMEM((_ZROWS, D), jnp.float32),
            pltpu.VMEM_SHARED((_NPAD, D), jnp.float32),
            pltpu.SemaphoreType.DMA,
        ],
        compiler_params=pltpu.CompilerParams(
            needs_layout_passes=False, use_tc_tiling_on_sc=tc_tiling),
    )


def _segment_sum_sc(g, row2d, col2d, D, tc_tiling=True):
    partials = _make_agg(D, tc_tiling)(g, row2d, col2d)
    return partials[0, :N_NODES] + partials[1, :N_NODES]


def kernel(x, edge_index, W1, b1, gamma, beta, W2, b2):
    row = edge_index[0]
    col = edge_index[1]
    row2d = row.reshape(_NW, _NCHUNK, _CHUNK)
    col2d = col.reshape(_NW, _NCHUNK, _CHUNK)

    deg = 1.0 + jnp.sum(_deg_partials(col), axis=0)
    dis = lax.rsqrt(deg)

    g1 = (x @ W1) * dis[:, None]
    agg1 = _segment_sum_sc(g1, row2d, col2d, 128)
    t = dis[:, None] * (agg1 + g1) + b1
    mean = jnp.mean(t, axis=0)
    var = jnp.var(t, axis=0)
    h = jax.nn.relu(gamma * (t - mean) / jnp.sqrt(var + BN_EPS) + beta)

    g2 = jnp.pad((h @ W2) * dis[:, None], ((0, 0), (0, 8)))
    agg2 = _segment_sum_sc(g2, row2d, col2d, 48, tc_tiling=False)
    out = dis[:, None] * (agg2[:, :40] + g2[:, :40]) + b2
    return out
